# baseline (speedup 1.0000x reference)
import functools

import jax
import jax.numpy as jnp
from jax import lax
from jax.experimental import pallas as pl
from jax.experimental.pallas import tpu as pltpu
from jax.experimental.pallas import tpu_sc as plsc

BATCH = 4
SEQ_LEN = 2048
D_MODEL = 768
_ROWS = BATCH * SEQ_LEN

_NUM_CORES = 2
_NUM_SUBCORES = 16
_NW = _NUM_CORES * _NUM_SUBCORES
_S_PER_W = SEQ_LEN // _NW
_HALF = _S_PER_W // 2
_NSUB = 4
_SUB = _HALF // _NSUB
_NHC = BATCH * 2
_LANES = 16
_D_SLICES = D_MODEL // _LANES


def _body(x_hbm, tok_hbm, pos_hbm, out_hbm, idx_v, tok_v, pos_v,
          g0, g1, g2, g3, ssem, psem):
    wid = lax.axis_index("s") * _NUM_CORES + lax.axis_index("c")
    s_base = wid * _S_PER_W
    gsems = (g0, g1, g2, g3)

    for b in range(BATCH):
        pltpu.sync_copy(x_hbm.at[b, pl.ds(s_base, _S_PER_W)],
                        idx_v.at[pl.ds(b * _S_PER_W, _S_PER_W)])

    def sub_gather(i, h):
        off = lax.rem(i, 2) * _HALF + h * _SUB
        return pltpu.make_async_copy(
            tok_hbm.at[idx_v.at[pl.ds(i * _HALF + h * _SUB, _SUB)]],
            tok_v.at[pl.ds(off, _SUB)], gsems[h])

    def sub_store(i, h):
        off = lax.rem(i, 2) * _HALF + h * _SUB
        row_base = (lax.div(i, 2) * SEQ_LEN + s_base
                    + lax.rem(i, 2) * _HALF + h * _SUB)
        return pltpu.make_async_copy(
            tok_v.at[pl.ds(off, _SUB)],
            out_hbm.at[pl.ds(row_base, _SUB)], ssem)

    def add_sub(i, h):
        off = lax.rem(i, 2) * _HALF + h * _SUB

        @plsc.parallel_loop(0, _SUB, step=1, unroll=2)
        def per_row(r):
            for j in range(_D_SLICES):
                sl = pl.ds(j * _LANES, _LANES)
                plsc.addupdate(tok_v.at[off + r, sl], pos_v[off + r, sl])

    for h in range(_NSUB):
        sub_gather(0, h).start()
    pos_cp = pltpu.make_async_copy(pos_hbm.at[pl.ds(s_base, _S_PER_W)],
                                   pos_v, psem)
    pos_cp.start()
    pos_cp.wait()

    def step(i, _):
        for h in range(_NSUB):
            sub_gather(i, h).wait()
            sub_gather(i + 1, h).start()
            add_sub(i, h)
            sub_store(i, h).start()
        for h in range(_NSUB):
            sub_store(i, h).wait()
        return 0

    lax.fori_loop(0, _NHC - 1, step, 0, unroll=False)

    last = _NHC - 1
    for h in range(_NSUB):
        sub_gather(last, h).wait()
        add_sub(last, h)
        sub_store(last, h).start()
    for h in range(_NSUB):
        sub_store(last, h).wait()


@functools.partial(
    pl.kernel,
    out_type=jax.ShapeDtypeStruct((_ROWS, D_MODEL), jnp.float32),
    mesh=plsc.VectorSubcoreMesh(core_axis_name="c", subcore_axis_name="s"),
    scratch_types=[
        pltpu.VMEM((_NHC * _HALF,), jnp.int32),
        pltpu.VMEM((_S_PER_W, D_MODEL), jnp.float32),
        pltpu.VMEM((_S_PER_W, D_MODEL), jnp.float32),
        pltpu.SemaphoreType.DMA,
        pltpu.SemaphoreType.DMA,
        pltpu.SemaphoreType.DMA,
        pltpu.SemaphoreType.DMA,
        pltpu.SemaphoreType.DMA,
        pltpu.SemaphoreType.DMA,
    ],
)
def _emb_lookup(x_hbm, tok_hbm, pos_hbm, out_hbm, idx_v, tok_v, pos_v,
                g0, g1, g2, g3, ssem, psem):
    _body(x_hbm, tok_hbm, pos_hbm, out_hbm, idx_v, tok_v, pos_v,
          g0, g1, g2, g3, ssem, psem)


def kernel(x, token_table, position_table):
    x = x.astype(jnp.int32)
    out = _emb_lookup(x, token_table, position_table)
    return out.reshape(BATCH, SEQ_LEN, D_MODEL)
